# baseline (speedup 1.0000x reference)
import functools

import jax
import jax.numpy as jnp
from jax import lax
from jax.experimental import pallas as pl
from jax.experimental.pallas import tpu as pltpu
from jax.experimental.pallas import tpu_sc as plsc

NC = 2
NS = 16
L = 16

D = 256
ECH = 128
ACH = 64
ACH_SH = 6
RING = 2
RING_SH = 7

_mesh = functools.partial(
    plsc.VectorSubcoreMesh,
    core_axis_name="c", subcore_axis_name="s", num_cores=NC, num_subcores=NS,
)


def _deg_kernel(npad, e_pad):
    ept = e_pad // (NC * NS)
    nchunk = ept // ECH
    rows_pt = npad // NS

    @functools.partial(
        pl.kernel,
        out_type=jax.ShapeDtypeStruct((NC, npad, L), jnp.float32),
        mesh=_mesh(),
        compiler_params=pltpu.CompilerParams(use_tc_tiling_on_sc=False, needs_layout_passes=False),
        scratch_types=[
            pltpu.VMEM_SHARED((npad, L), jnp.float32),
            pltpu.VMEM((ECH,), jnp.int32),
            pltpu.VMEM((ECH, L), jnp.float32),
        ],
    )
    def deg(dst_hbm, zeros_hbm, ones_hbm, out_hbm, acc_sh, idx_v, ones_v):
        c = lax.axis_index("c")
        t = lax.axis_index("s")
        wid = c * NS + t
        pltpu.sync_copy(zeros_hbm, acc_sh.at[pl.ds(t * rows_pt, rows_pt)])
        pltpu.sync_copy(ones_hbm, ones_v)
        plsc.subcore_barrier()

        def chunk(i, carry):
            base = wid * ept + i * ECH
            pltpu.sync_copy(dst_hbm.at[pl.ds(base, ECH)], idx_v)
            pltpu.sync_copy(ones_v, acc_sh.at[idx_v], add=True)
            return carry

        lax.fori_loop(0, nchunk, chunk, 0)
        plsc.subcore_barrier()
        pltpu.sync_copy(acc_sh.at[pl.ds(t * rows_pt, rows_pt)],
                        out_hbm.at[c, pl.ds(t * rows_pt, rows_pt)])

    return deg


def _prep_kernel(npad, e_pad):
    half = npad // NC
    ept = e_pad // NS
    seg_ch = ept // ACH
    stg = 512
    nstage = ept // stg

    @functools.partial(
        pl.kernel,
        out_type=(
            jax.ShapeDtypeStruct((NC, NS, seg_ch, ACH), jnp.int32),
            jax.ShapeDtypeStruct((NC, NS, seg_ch, ACH), jnp.int32),
            jax.ShapeDtypeStruct((NC, NS, L), jnp.int32),
        ),
        mesh=_mesh(),
        compiler_params=pltpu.CompilerParams(use_tc_tiling_on_sc=False, needs_layout_passes=False),
        scratch_types=[
            pltpu.VMEM((stg,), jnp.int32),
            pltpu.VMEM((stg,), jnp.int32),
            pltpu.VMEM((seg_ch, ACH), jnp.int32),
            pltpu.VMEM((seg_ch, ACH), jnp.int32),
            pltpu.VMEM((L,), jnp.int32),
        ],
    )
    def prep(src_hbm, dst_hbm, zseg_hbm, dseg_hbm, csrc_hbm, cdloc_hbm,
             ccnt_hbm, stage_s, stage_d, segs_v, segd_v, cnt_v):
        c = lax.axis_index("c")
        t = lax.axis_index("s")
        cbase = c * half
        pltpu.sync_copy(zseg_hbm, segs_v)
        pltpu.sync_copy(dseg_hbm, segd_v)

        def scan(g, off_vec):
            base = t * ept + g * stg
            pltpu.sync_copy(src_hbm.at[pl.ds(base, stg)], stage_s)
            pltpu.sync_copy(dst_hbm.at[pl.ds(base, stg)], stage_d)
            for u in range(stg // L):
                sv = stage_s[pl.ds(u * L, L)]
                dv = stage_d[pl.ds(u * L, L)]
                d = dv - cbase
                m = (d >= 0) & (d < half)
                pos = off_vec + plsc.cumsum(m.astype(jnp.int32)) - 1
                plsc.store_scatter(segs_v, [pos >> ACH_SH, pos & (ACH - 1)],
                                   sv, mask=m)
                plsc.store_scatter(segd_v, [pos >> ACH_SH, pos & (ACH - 1)],
                                   d, mask=m)
                off_vec = off_vec + plsc.all_reduce_population_count(m)
            return off_vec

        off_vec = lax.fori_loop(0, nstage, scan, jnp.zeros((L,), jnp.int32))
        cnt_v[...] = off_vec
        pltpu.sync_copy(segs_v, csrc_hbm.at[c, t])
        pltpu.sync_copy(segd_v, cdloc_hbm.at[c, t])
        pltpu.sync_copy(cnt_v, ccnt_hbm.at[c, t])

    return prep


def _acc_kernel(npad, e_pad):
    half = npad // NC
    rows_pt = half // NS
    ept = e_pad // NS
    seg_ch = ept // ACH
    hch = seg_ch // 2
    n_dummy = (NS + 7) // 8 * 8

    @functools.partial(
        pl.kernel,
        out_type=jax.ShapeDtypeStruct((npad, D), jnp.float32),
        mesh=_mesh(),
        compiler_params=pltpu.CompilerParams(use_tc_tiling_on_sc=False, needs_layout_passes=False),
        scratch_types=[
            pltpu.VMEM_SHARED((half + n_dummy, D), jnp.float32),
            pltpu.VMEM((hch, ACH), jnp.int32),
            pltpu.VMEM((hch, ACH), jnp.int32),
        ] + [pltpu.VMEM((ACH, D), jnp.float32) for _ in range(RING)]
        + [pltpu.VMEM((L,), jnp.int32)]
        + [pltpu.SemaphoreType.DMA for _ in range(2 * RING)],
    )
    def acc(y_hbm, csrc_hbm, cdloc_hbm, ccnt_hbm, out_hbm,
            acc_sh, sidx_v, didx_v, *rest):
        rows = rest[:RING]
        cnt_v = rest[RING]
        gsem = rest[RING + 1:2 * RING + 1]
        ssem = rest[2 * RING + 1:]
        c = lax.axis_index("c")
        t = lax.axis_index("s")
        base_row = c * half + t * rows_pt
        pltpu.sync_copy(y_hbm.at[pl.ds(base_row, rows_pt)],
                        acc_sh.at[pl.ds(t * rows_pt, rows_pt)])
        pltpu.sync_copy(ccnt_hbm.at[c, t], cnt_v)
        plsc.subcore_barrier()
        cnt = jnp.max(cnt_v[...])

        for h in range(2):
            pltpu.sync_copy(csrc_hbm.at[c, t, pl.ds(h * hch, hch)], sidx_v)
            pltpu.sync_copy(cdloc_hbm.at[c, t, pl.ds(h * hch, hch)], didx_v)
            rem = jnp.clip(cnt - h * (hch * ACH), 0, hch * ACH)
            nring = (rem + RING * ACH - 1) >> RING_SH

            def ring_iter(j, carry):
                for s in range(RING):
                    k = RING * j + s

                    @pl.when(j > 0)
                    def _():
                        pltpu.make_async_copy(
                            rows[s], acc_sh.at[didx_v.at[0]], ssem[s]).wait()
                    pltpu.async_copy(y_hbm.at[sidx_v.at[k]], rows[s], gsem[s])
                for s in range(RING):
                    k = RING * j + s
                    pltpu.make_async_copy(y_hbm.at[sidx_v.at[k]], rows[s],
                                          gsem[s]).wait()
                    pltpu.async_copy(rows[s], acc_sh.at[didx_v.at[k]],
                                     ssem[s], add=True)
                return carry

            lax.fori_loop(0, nring, ring_iter, 0)

            @pl.when(nring > 0)
            def _():
                for s in range(RING):
                    pltpu.make_async_copy(rows[s], acc_sh.at[didx_v.at[0]],
                                          ssem[s]).wait()

        plsc.subcore_barrier()
        pltpu.sync_copy(acc_sh.at[pl.ds(t * rows_pt, rows_pt)],
                        out_hbm.at[pl.ds(base_row, rows_pt)])

    return acc


def _tc_in_stage(npad, din, dh, br):
    def body(x_ref, w_ref, d0_ref, d1_ref, o_ref):
        dinv = lax.rsqrt(d0_ref[...] + d1_ref[...] + 1.0)
        o_ref[...] = jnp.dot(x_ref[...], w_ref[...],
                             preferred_element_type=jnp.float32) * dinv

    return pl.pallas_call(
        body,
        out_shape=jax.ShapeDtypeStruct((npad, dh), jnp.float32),
        grid=(npad // br,),
        in_specs=[
            pl.BlockSpec((br, din), lambda i: (i, 0)),
            pl.BlockSpec((din, dh), lambda i: (0, 0)),
            pl.BlockSpec((br, 1), lambda i: (i, 0)),
            pl.BlockSpec((br, 1), lambda i: (i, 0)),
        ],
        out_specs=pl.BlockSpec((br, dh), lambda i: (i, 0)),
    )


def _tc_mid_stage(npad, dh, br):
    def body(s_ref, d0_ref, d1_ref, b_ref, o_ref):
        dinv = lax.rsqrt(d0_ref[...] + d1_ref[...] + 1.0)
        h = jnp.maximum(s_ref[...] * dinv + b_ref[...], 0.0)
        o_ref[...] = h * dinv

    return pl.pallas_call(
        body,
        out_shape=jax.ShapeDtypeStruct((npad, dh), jnp.float32),
        grid=(npad // br,),
        in_specs=[
            pl.BlockSpec((br, dh), lambda i: (i, 0)),
            pl.BlockSpec((br, 1), lambda i: (i, 0)),
            pl.BlockSpec((br, 1), lambda i: (i, 0)),
            pl.BlockSpec((1, dh), lambda i: (0, 0)),
        ],
        out_specs=pl.BlockSpec((br, dh), lambda i: (i, 0)),
    )


def _tc_out_stage(npad, dh, dout, br):
    def body(s_ref, w_ref, d0_ref, d1_ref, b_ref, o_ref):
        dinv = lax.rsqrt(d0_ref[...] + d1_ref[...] + 1.0)
        o_ref[...] = jnp.dot(s_ref[...] * dinv, w_ref[...],
                             preferred_element_type=jnp.float32) + b_ref[...]

    return pl.pallas_call(
        body,
        out_shape=jax.ShapeDtypeStruct((npad, dout), jnp.float32),
        grid=(npad // br,),
        in_specs=[
            pl.BlockSpec((br, dh), lambda i: (i, 0)),
            pl.BlockSpec((dh, dout), lambda i: (0, 0)),
            pl.BlockSpec((br, 1), lambda i: (i, 0)),
            pl.BlockSpec((br, 1), lambda i: (i, 0)),
            pl.BlockSpec((1, dout), lambda i: (0, 0)),
        ],
        out_specs=pl.BlockSpec((br, dout), lambda i: (i, 0)),
    )


def kernel(x, edge_index, W1, b1, W2, b2):
    n, din = x.shape
    dh = W1.shape[1]
    dout = W2.shape[1]
    e = edge_index.shape[1]

    npad = ((n + (NC * NS * 8) - 1) // (NC * NS * 8)) * (NC * NS * 8)
    if npad == n:
        npad += NC * NS * 8
    epg = NC * NS * ECH
    e_pad = ((e + epg - 1) // epg) * epg

    src = edge_index[0].astype(jnp.int32)
    dst = edge_index[1].astype(jnp.int32)
    src_p = jnp.concatenate(
        [src, jnp.arange(e_pad - e, dtype=jnp.int32) % n])
    dst_p = jnp.concatenate(
        [dst, n + jnp.arange(e_pad - e, dtype=jnp.int32) % (npad - n)])
    xp = jnp.pad(x, ((0, npad - n), (0, 0)))

    zeros_slab = jnp.zeros((npad // NS, L), jnp.float32)
    ones_chunk = jnp.ones((ECH, L), jnp.float32)
    seg_ch = (e_pad // NS) // ACH
    zseg = jnp.zeros((seg_ch, ACH), jnp.int32)
    dseg = jnp.full((seg_ch, ACH), npad // NC, jnp.int32)

    degp = _deg_kernel(npad, e_pad)(dst_p, zeros_slab, ones_chunk)
    csrc, cdloc, ccnt = _prep_kernel(npad, e_pad)(src_p, dst_p, zseg, dseg)
    d0 = degp[0, :, 0:1]
    d1 = degp[1, :, 0:1]

    acc = _acc_kernel(npad, e_pad)
    y1 = _tc_in_stage(npad, din, dh, 1024)(xp, W1, d0, d1)
    s1 = acc(y1, csrc, cdloc, ccnt)
    z = _tc_mid_stage(npad, dh, 1024)(s1, d0, d1, b1.reshape(1, dh))
    s2 = acc(z, csrc, cdloc, ccnt)
    out = _tc_out_stage(npad, dh, dout, 512)(s2, W2, d0, d1, b2.reshape(1, dout))
    return out[:n]


# speedup vs baseline: 1.1654x; 1.1654x over previous
import functools

import jax
import jax.numpy as jnp
from jax import lax
from jax.experimental import pallas as pl
from jax.experimental.pallas import tpu as pltpu
from jax.experimental.pallas import tpu_sc as plsc

NC = 2
NS = 16
L = 16

D = 256
ECH = 128
ACH = 32
ACH_SH = 5
RING = 4
RING_SH = 7

_mesh = functools.partial(
    plsc.VectorSubcoreMesh,
    core_axis_name="c", subcore_axis_name="s", num_cores=NC, num_subcores=NS,
)


def _prep_kernel(npad, e_pad):
    half = npad // NC
    rows_pt = half // NS
    ept = e_pad // NS
    seg_ch = ept // ACH
    stg = 512
    nstage = ept // stg

    @functools.partial(
        pl.kernel,
        out_type=(
            jax.ShapeDtypeStruct((NC, NS, seg_ch, ACH), jnp.int32),
            jax.ShapeDtypeStruct((NC, NS, seg_ch, ACH), jnp.int32),
            jax.ShapeDtypeStruct((NC, NS, L), jnp.int32),
            jax.ShapeDtypeStruct((npad, L), jnp.float32),
        ),
        mesh=_mesh(),
        compiler_params=pltpu.CompilerParams(use_tc_tiling_on_sc=False, needs_layout_passes=False),
        scratch_types=[
            pltpu.VMEM_SHARED((npad, L), jnp.float32),
            pltpu.VMEM((stg,), jnp.int32),
            pltpu.VMEM((stg // ECH, ECH), jnp.int32),
            pltpu.VMEM((seg_ch, ACH), jnp.int32),
            pltpu.VMEM((seg_ch, ACH), jnp.int32),
            pltpu.VMEM((L,), jnp.int32),
            pltpu.VMEM((ECH, L), jnp.float32),
            pltpu.VMEM((npad // NS, L), jnp.float32),
        ],
    )
    def prep(src_hbm, dst2_hbm, zeros_hbm, ones_hbm, zseg_hbm, dseg_hbm,
             csrc_hbm, cdloc_hbm, ccnt_hbm, deg_hbm,
             deg_sh, stage_s, stage_d, segs_v, segd_v, cnt_v, ones_v, zslab_v):
        c = lax.axis_index("c")
        t = lax.axis_index("s")
        cbase = c * half
        pltpu.sync_copy(zseg_hbm, segs_v)
        pltpu.sync_copy(dseg_hbm, segd_v)
        pltpu.sync_copy(zeros_hbm, zslab_v)
        pltpu.sync_copy(zslab_v, deg_sh.at[pl.ds(t * (npad // NS), npad // NS)])
        pltpu.sync_copy(ones_hbm, ones_v)
        plsc.subcore_barrier()

        def scan(g, off_vec):
            base = t * ept + g * stg
            pltpu.sync_copy(src_hbm.at[pl.ds(base, stg)], stage_s)
            pltpu.sync_copy(dst2_hbm.at[pl.ds(base // ECH, stg // ECH)],
                            stage_d)
            for q in range(stg // ECH):
                pltpu.sync_copy(ones_v, deg_sh.at[stage_d.at[q]], add=True)
            for u in range(stg // L):
                sv = stage_s[pl.ds(u * L, L)]
                dv = stage_d[u // (ECH // L), pl.ds((u % (ECH // L)) * L, L)]
                d = dv - cbase
                m = (d >= 0) & (d < half)
                pos = off_vec + plsc.cumsum(m.astype(jnp.int32)) - 1
                plsc.store_scatter(segs_v, [pos >> ACH_SH, pos & (ACH - 1)],
                                   sv, mask=m)
                plsc.store_scatter(segd_v, [pos >> ACH_SH, pos & (ACH - 1)],
                                   d, mask=m)
                off_vec = off_vec + plsc.all_reduce_population_count(m)
            return off_vec

        off_vec = lax.fori_loop(0, nstage, scan, jnp.zeros((L,), jnp.int32))
        cnt_v[...] = off_vec
        pltpu.sync_copy(segs_v, csrc_hbm.at[c, t])
        pltpu.sync_copy(segd_v, cdloc_hbm.at[c, t])
        pltpu.sync_copy(cnt_v, ccnt_hbm.at[c, t])
        plsc.subcore_barrier()
        pltpu.sync_copy(deg_sh.at[pl.ds(cbase + t * rows_pt, rows_pt)],
                        deg_hbm.at[pl.ds(cbase + t * rows_pt, rows_pt)])

    return prep


def _acc_kernel(npad, e_pad):
    half = npad // NC
    rows_pt = half // NS
    ept = e_pad // NS
    seg_ch = ept // ACH
    hch = seg_ch // 2
    n_dummy = (NS + 7) // 8 * 8

    @functools.partial(
        pl.kernel,
        out_type=jax.ShapeDtypeStruct((npad, D), jnp.float32),
        mesh=_mesh(),
        compiler_params=pltpu.CompilerParams(use_tc_tiling_on_sc=False, needs_layout_passes=False),
        scratch_types=[
            pltpu.VMEM_SHARED((half + n_dummy, D), jnp.float32),
            pltpu.VMEM((hch, ACH), jnp.int32),
            pltpu.VMEM((hch, ACH), jnp.int32),
        ] + [pltpu.VMEM((ACH, D), jnp.float32) for _ in range(RING)]
        + [pltpu.VMEM((L,), jnp.int32)]
        + [pltpu.SemaphoreType.DMA for _ in range(2 * RING)],
    )
    def acc(y_hbm, csrc_hbm, cdloc_hbm, ccnt_hbm, out_hbm,
            acc_sh, sidx_v, didx_v, *rest):
        rows = rest[:RING]
        cnt_v = rest[RING]
        gsem = rest[RING + 1:2 * RING + 1]
        ssem = rest[2 * RING + 1:]
        c = lax.axis_index("c")
        t = lax.axis_index("s")
        base_row = c * half + t * rows_pt
        pltpu.sync_copy(y_hbm.at[pl.ds(base_row, rows_pt)],
                        acc_sh.at[pl.ds(t * rows_pt, rows_pt)])
        pltpu.sync_copy(ccnt_hbm.at[c, t], cnt_v)
        plsc.subcore_barrier()
        cnt = jnp.max(cnt_v[...])

        for h in range(2):
            pltpu.sync_copy(csrc_hbm.at[c, t, pl.ds(h * hch, hch)], sidx_v)
            pltpu.sync_copy(cdloc_hbm.at[c, t, pl.ds(h * hch, hch)], didx_v)
            rem = jnp.clip(cnt - h * (hch * ACH), 0, hch * ACH)
            nring = (rem + RING * ACH - 1) >> RING_SH

            def ring_iter(j, carry):
                for s in range(RING):
                    k = RING * j + s

                    @pl.when(j > 0)
                    def _():
                        pltpu.make_async_copy(
                            rows[s], acc_sh.at[didx_v.at[0]], ssem[s]).wait()
                    pltpu.async_copy(y_hbm.at[sidx_v.at[k]], rows[s], gsem[s])
                for s in range(RING):
                    k = RING * j + s
                    pltpu.make_async_copy(y_hbm.at[sidx_v.at[k]], rows[s],
                                          gsem[s]).wait()
                    pltpu.async_copy(rows[s], acc_sh.at[didx_v.at[k]],
                                     ssem[s], add=True)
                return carry

            lax.fori_loop(0, nring, ring_iter, 0)

            @pl.when(nring > 0)
            def _():
                for s in range(RING):
                    pltpu.make_async_copy(rows[s], acc_sh.at[didx_v.at[0]],
                                          ssem[s]).wait()

        plsc.subcore_barrier()
        pltpu.sync_copy(acc_sh.at[pl.ds(t * rows_pt, rows_pt)],
                        out_hbm.at[pl.ds(base_row, rows_pt)])

    return acc


def _tc_in_stage(npad, din, dh, br):
    def body(x_ref, w_ref, dg_ref, o_ref):
        dinv = lax.rsqrt(dg_ref[...] + 1.0)
        o_ref[...] = jnp.dot(x_ref[...], w_ref[...],
                             preferred_element_type=jnp.float32) * dinv

    return pl.pallas_call(
        body,
        out_shape=jax.ShapeDtypeStruct((npad, dh), jnp.float32),
        grid=(npad // br,),
        in_specs=[
            pl.BlockSpec((br, din), lambda i: (i, 0)),
            pl.BlockSpec((din, dh), lambda i: (0, 0)),
            pl.BlockSpec((br, 1), lambda i: (i, 0)),
        ],
        out_specs=pl.BlockSpec((br, dh), lambda i: (i, 0)),
    )


def _tc_mid_stage(npad, dh, br):
    def body(s_ref, dg_ref, b_ref, o_ref):
        dinv = lax.rsqrt(dg_ref[...] + 1.0)
        h = jnp.maximum(s_ref[...] * dinv + b_ref[...], 0.0)
        o_ref[...] = h * dinv

    return pl.pallas_call(
        body,
        out_shape=jax.ShapeDtypeStruct((npad, dh), jnp.float32),
        grid=(npad // br,),
        in_specs=[
            pl.BlockSpec((br, dh), lambda i: (i, 0)),
            pl.BlockSpec((br, 1), lambda i: (i, 0)),
            pl.BlockSpec((1, dh), lambda i: (0, 0)),
        ],
        out_specs=pl.BlockSpec((br, dh), lambda i: (i, 0)),
    )


def _tc_out_stage(npad, dh, dout, br):
    def body(s_ref, w_ref, dg_ref, b_ref, o_ref):
        dinv = lax.rsqrt(dg_ref[...] + 1.0)
        o_ref[...] = jnp.dot(s_ref[...] * dinv, w_ref[...],
                             preferred_element_type=jnp.float32) + b_ref[...]

    return pl.pallas_call(
        body,
        out_shape=jax.ShapeDtypeStruct((npad, dout), jnp.float32),
        grid=(npad // br,),
        in_specs=[
            pl.BlockSpec((br, dh), lambda i: (i, 0)),
            pl.BlockSpec((dh, dout), lambda i: (0, 0)),
            pl.BlockSpec((br, 1), lambda i: (i, 0)),
            pl.BlockSpec((1, dout), lambda i: (0, 0)),
        ],
        out_specs=pl.BlockSpec((br, dout), lambda i: (i, 0)),
    )


def kernel(x, edge_index, W1, b1, W2, b2):
    n, din = x.shape
    dh = W1.shape[1]
    dout = W2.shape[1]
    e = edge_index.shape[1]

    npad = ((n + (NC * NS * 8) - 1) // (NC * NS * 8)) * (NC * NS * 8)
    if npad == n:
        npad += NC * NS * 8
    epg = NC * NS * ECH
    e_pad = ((e + epg - 1) // epg) * epg

    src = edge_index[0].astype(jnp.int32)
    dst = edge_index[1].astype(jnp.int32)
    src_p = jnp.concatenate(
        [src, jnp.arange(e_pad - e, dtype=jnp.int32) % n])
    dst_p = jnp.concatenate(
        [dst, n + jnp.arange(e_pad - e, dtype=jnp.int32) % (npad - n)])
    xp = jnp.pad(x, ((0, npad - n), (0, 0)))

    zeros_slab = jnp.zeros((npad // NS, L), jnp.float32)
    ones_chunk = jnp.ones((ECH, L), jnp.float32)
    seg_ch = (e_pad // NS) // ACH
    zseg = jnp.zeros((seg_ch, ACH), jnp.int32)
    dseg = jnp.full((seg_ch, ACH), npad // NC, jnp.int32)

    dst2 = dst_p.reshape(e_pad // ECH, ECH)
    csrc, cdloc, ccnt, degf = _prep_kernel(npad, e_pad)(
        src_p, dst2, zeros_slab, ones_chunk, zseg, dseg)
    dd = degf[:, 0:1]

    acc = _acc_kernel(npad, e_pad)
    y1 = _tc_in_stage(npad, din, dh, 1024)(xp, W1, dd)
    s1 = acc(y1, csrc, cdloc, ccnt)
    z = _tc_mid_stage(npad, dh, 1024)(s1, dd, b1.reshape(1, dh))
    s2 = acc(z, csrc, cdloc, ccnt)
    out = _tc_out_stage(npad, dh, dout, 512)(s2, W2, dd, b2.reshape(1, dout))
    return out[:n]


# speedup vs baseline: 1.1823x; 1.0145x over previous
import functools

import jax
import jax.numpy as jnp
from jax import lax
from jax.experimental import pallas as pl
from jax.experimental.pallas import tpu as pltpu
from jax.experimental.pallas import tpu_sc as plsc

NC = 2
NS = 16
L = 16

D = 256
ECH = 128
ACH = 16
ACH_SH = 4
RING = 8
RING_SH = 7

_mesh = functools.partial(
    plsc.VectorSubcoreMesh,
    core_axis_name="c", subcore_axis_name="s", num_cores=NC, num_subcores=NS,
)


def _prep_kernel(npad, e_pad):
    half = npad // NC
    rows_pt = half // NS
    ept = e_pad // NS
    seg_ch = ept // ACH
    stg = 512
    nstage = ept // stg

    @functools.partial(
        pl.kernel,
        out_type=(
            jax.ShapeDtypeStruct((NC, NS, seg_ch, ACH), jnp.int32),
            jax.ShapeDtypeStruct((NC, NS, seg_ch, ACH), jnp.int32),
            jax.ShapeDtypeStruct((NC, NS, L), jnp.int32),
            jax.ShapeDtypeStruct((npad, L), jnp.float32),
        ),
        mesh=_mesh(),
        compiler_params=pltpu.CompilerParams(use_tc_tiling_on_sc=False, needs_layout_passes=False),
        scratch_types=[
            pltpu.VMEM_SHARED((npad, L), jnp.float32),
            pltpu.VMEM((stg,), jnp.int32),
            pltpu.VMEM((stg // ECH, ECH), jnp.int32),
            pltpu.VMEM((seg_ch, ACH), jnp.int32),
            pltpu.VMEM((seg_ch, ACH), jnp.int32),
            pltpu.VMEM((L,), jnp.int32),
            pltpu.VMEM((ECH, L), jnp.float32),
            pltpu.VMEM((npad // NS, L), jnp.float32),
        ],
    )
    def prep(src_hbm, dst2_hbm, zeros_hbm, ones_hbm, zseg_hbm, dseg_hbm,
             csrc_hbm, cdloc_hbm, ccnt_hbm, deg_hbm,
             deg_sh, stage_s, stage_d, segs_v, segd_v, cnt_v, ones_v, zslab_v):
        c = lax.axis_index("c")
        t = lax.axis_index("s")
        cbase = c * half
        pltpu.sync_copy(zseg_hbm, segs_v)
        pltpu.sync_copy(dseg_hbm, segd_v)
        pltpu.sync_copy(zeros_hbm, zslab_v)
        pltpu.sync_copy(zslab_v, deg_sh.at[pl.ds(t * (npad // NS), npad // NS)])
        pltpu.sync_copy(ones_hbm, ones_v)
        plsc.subcore_barrier()

        def scan(g, off_vec):
            base = t * ept + g * stg
            pltpu.sync_copy(src_hbm.at[pl.ds(base, stg)], stage_s)
            pltpu.sync_copy(dst2_hbm.at[pl.ds(base // ECH, stg // ECH)],
                            stage_d)
            for q in range(stg // ECH):
                pltpu.sync_copy(ones_v, deg_sh.at[stage_d.at[q]], add=True)
            for u in range(stg // L):
                sv = stage_s[pl.ds(u * L, L)]
                dv = stage_d[u // (ECH // L), pl.ds((u % (ECH // L)) * L, L)]
                d = dv - cbase
                m = (d >= 0) & (d < half)
                pos = off_vec + plsc.cumsum(m.astype(jnp.int32)) - 1
                plsc.store_scatter(segs_v, [pos >> ACH_SH, pos & (ACH - 1)],
                                   sv, mask=m)
                plsc.store_scatter(segd_v, [pos >> ACH_SH, pos & (ACH - 1)],
                                   d, mask=m)
                off_vec = off_vec + plsc.all_reduce_population_count(m)
            return off_vec

        off_vec = lax.fori_loop(0, nstage, scan, jnp.zeros((L,), jnp.int32))
        cnt_v[...] = off_vec
        pltpu.sync_copy(segs_v, csrc_hbm.at[c, t])
        pltpu.sync_copy(segd_v, cdloc_hbm.at[c, t])
        pltpu.sync_copy(cnt_v, ccnt_hbm.at[c, t])
        plsc.subcore_barrier()
        pltpu.sync_copy(deg_sh.at[pl.ds(cbase + t * rows_pt, rows_pt)],
                        deg_hbm.at[pl.ds(cbase + t * rows_pt, rows_pt)])

    return prep


def _acc_kernel(npad, e_pad):
    half = npad // NC
    rows_pt = half // NS
    ept = e_pad // NS
    seg_ch = ept // ACH
    hch = seg_ch // 2
    n_dummy = (NS + 7) // 8 * 8

    @functools.partial(
        pl.kernel,
        out_type=jax.ShapeDtypeStruct((npad, D), jnp.float32),
        mesh=_mesh(),
        compiler_params=pltpu.CompilerParams(use_tc_tiling_on_sc=False, needs_layout_passes=False),
        scratch_types=[
            pltpu.VMEM_SHARED((half + n_dummy, D), jnp.float32),
            pltpu.VMEM((hch, ACH), jnp.int32),
            pltpu.VMEM((hch, ACH), jnp.int32),
        ] + [pltpu.VMEM((ACH, D), jnp.float32) for _ in range(RING)]
        + [pltpu.VMEM((L,), jnp.int32)]
        + [pltpu.SemaphoreType.DMA for _ in range(2 * RING)],
    )
    def acc(y_hbm, csrc_hbm, cdloc_hbm, ccnt_hbm, out_hbm,
            acc_sh, sidx_v, didx_v, *rest):
        rows = rest[:RING]
        cnt_v = rest[RING]
        gsem = rest[RING + 1:2 * RING + 1]
        ssem = rest[2 * RING + 1:]
        c = lax.axis_index("c")
        t = lax.axis_index("s")
        base_row = c * half + t * rows_pt
        pltpu.sync_copy(y_hbm.at[pl.ds(base_row, rows_pt)],
                        acc_sh.at[pl.ds(t * rows_pt, rows_pt)])
        pltpu.sync_copy(ccnt_hbm.at[c, t], cnt_v)
        plsc.subcore_barrier()
        cnt = jnp.max(cnt_v[...])

        for h in range(2):
            pltpu.sync_copy(csrc_hbm.at[c, t, pl.ds(h * hch, hch)], sidx_v)
            pltpu.sync_copy(cdloc_hbm.at[c, t, pl.ds(h * hch, hch)], didx_v)
            rem = jnp.clip(cnt - h * (hch * ACH), 0, hch * ACH)
            nring = (rem + RING * ACH - 1) >> RING_SH

            def ring_iter(j, carry):
                for s in range(RING):
                    k = RING * j + s

                    @pl.when(j > 0)
                    def _():
                        pltpu.make_async_copy(
                            rows[s], acc_sh.at[didx_v.at[0]], ssem[s]).wait()
                    pltpu.async_copy(y_hbm.at[sidx_v.at[k]], rows[s], gsem[s])
                for s in range(RING):
                    k = RING * j + s
                    pltpu.make_async_copy(y_hbm.at[sidx_v.at[k]], rows[s],
                                          gsem[s]).wait()
                    pltpu.async_copy(rows[s], acc_sh.at[didx_v.at[k]],
                                     ssem[s], add=True)
                return carry

            lax.fori_loop(0, nring, ring_iter, 0)

            @pl.when(nring > 0)
            def _():
                for s in range(RING):
                    pltpu.make_async_copy(rows[s], acc_sh.at[didx_v.at[0]],
                                          ssem[s]).wait()

        plsc.subcore_barrier()
        pltpu.sync_copy(acc_sh.at[pl.ds(t * rows_pt, rows_pt)],
                        out_hbm.at[pl.ds(base_row, rows_pt)])

    return acc


def _tc_in_stage(npad, din, dh, br):
    def body(x_ref, w_ref, dg_ref, o_ref):
        dinv = lax.rsqrt(dg_ref[...] + 1.0)
        o_ref[...] = jnp.dot(x_ref[...], w_ref[...],
                             preferred_element_type=jnp.float32) * dinv

    return pl.pallas_call(
        body,
        out_shape=jax.ShapeDtypeStruct((npad, dh), jnp.float32),
        grid=(npad // br,),
        in_specs=[
            pl.BlockSpec((br, din), lambda i: (i, 0)),
            pl.BlockSpec((din, dh), lambda i: (0, 0)),
            pl.BlockSpec((br, 1), lambda i: (i, 0)),
        ],
        out_specs=pl.BlockSpec((br, dh), lambda i: (i, 0)),
    )


def _tc_mid_stage(npad, dh, br):
    def body(s_ref, dg_ref, b_ref, o_ref):
        dinv = lax.rsqrt(dg_ref[...] + 1.0)
        h = jnp.maximum(s_ref[...] * dinv + b_ref[...], 0.0)
        o_ref[...] = h * dinv

    return pl.pallas_call(
        body,
        out_shape=jax.ShapeDtypeStruct((npad, dh), jnp.float32),
        grid=(npad // br,),
        in_specs=[
            pl.BlockSpec((br, dh), lambda i: (i, 0)),
            pl.BlockSpec((br, 1), lambda i: (i, 0)),
            pl.BlockSpec((1, dh), lambda i: (0, 0)),
        ],
        out_specs=pl.BlockSpec((br, dh), lambda i: (i, 0)),
    )


def _tc_out_stage(npad, dh, dout, br):
    def body(s_ref, w_ref, dg_ref, b_ref, o_ref):
        dinv = lax.rsqrt(dg_ref[...] + 1.0)
        o_ref[...] = jnp.dot(s_ref[...] * dinv, w_ref[...],
                             preferred_element_type=jnp.float32) + b_ref[...]

    return pl.pallas_call(
        body,
        out_shape=jax.ShapeDtypeStruct((npad, dout), jnp.float32),
        grid=(npad // br,),
        in_specs=[
            pl.BlockSpec((br, dh), lambda i: (i, 0)),
            pl.BlockSpec((dh, dout), lambda i: (0, 0)),
            pl.BlockSpec((br, 1), lambda i: (i, 0)),
            pl.BlockSpec((1, dout), lambda i: (0, 0)),
        ],
        out_specs=pl.BlockSpec((br, dout), lambda i: (i, 0)),
    )


def kernel(x, edge_index, W1, b1, W2, b2):
    n, din = x.shape
    dh = W1.shape[1]
    dout = W2.shape[1]
    e = edge_index.shape[1]

    npad = ((n + (NC * NS * 8) - 1) // (NC * NS * 8)) * (NC * NS * 8)
    if npad == n:
        npad += NC * NS * 8
    epg = NC * NS * ECH
    e_pad = ((e + epg - 1) // epg) * epg

    src = edge_index[0].astype(jnp.int32)
    dst = edge_index[1].astype(jnp.int32)
    src_p = jnp.concatenate(
        [src, jnp.arange(e_pad - e, dtype=jnp.int32) % n])
    dst_p = jnp.concatenate(
        [dst, n + jnp.arange(e_pad - e, dtype=jnp.int32) % (npad - n)])
    xp = jnp.pad(x, ((0, npad - n), (0, 0)))

    zeros_slab = jnp.zeros((npad // NS, L), jnp.float32)
    ones_chunk = jnp.ones((ECH, L), jnp.float32)
    seg_ch = (e_pad // NS) // ACH
    zseg = jnp.zeros((seg_ch, ACH), jnp.int32)
    dseg = jnp.full((seg_ch, ACH), npad // NC, jnp.int32)

    dst2 = dst_p.reshape(e_pad // ECH, ECH)
    csrc, cdloc, ccnt, degf = _prep_kernel(npad, e_pad)(
        src_p, dst2, zeros_slab, ones_chunk, zseg, dseg)
    dd = degf[:, 0:1]

    acc = _acc_kernel(npad, e_pad)
    y1 = _tc_in_stage(npad, din, dh, 1024)(xp, W1, dd)
    s1 = acc(y1, csrc, cdloc, ccnt)
    z = _tc_mid_stage(npad, dh, 1024)(s1, dd, b1.reshape(1, dh))
    s2 = acc(z, csrc, cdloc, ccnt)
    out = _tc_out_stage(npad, dh, dout, 512)(s2, W2, dd, b2.reshape(1, dout))
    return out[:n]
